# baseline (speedup 1.0000x reference)
import functools

import jax
import jax.numpy as jnp
from jax import lax
from jax.experimental import pallas as pl
from jax.experimental.pallas import tpu as pltpu
from jax.experimental.pallas import tpu_sc as plsc

BATCH = 1024
IN_DIM = 4096
OUT_DIM = 16384

NC = 2
NS = 16
NW = NC * NS

N_BLOCKS = 2
R_BLOCKS = NW // N_BLOCKS
NCHUNK = OUT_DIM // N_BLOCKS
ROWS_PER_TILE = BATCH // R_BLOCKS
RG = 16
RGP = RG // 2
NGROUPS = ROWS_PER_TILE // RG
OCW = 1024
NOC = NCHUNK // OCW
NVEC = OCW // 16
CFS = 2 * NCHUNK


def _coeff_body(wt_ref, out_ref):
    w = wt_ref[...]
    m = jnp.max(w, axis=0, keepdims=True)
    e = jnp.exp(w - m)
    p = e / jnp.sum(e, axis=0, keepdims=True)

    def r(i):
        return p[i:i + 1, :]

    w0 = r(8) + r(9) + r(10) + r(11) + r(12) + r(13) + r(14) + r(15)
    wa = r(2) + r(3) + r(6) + r(7) - r(8) - r(9) - r(12) - r(13)
    wb = r(4) + r(5) + r(6) + r(7) - r(8) - r(9) - r(10) - r(11)
    wab = (r(1) - r(2) - r(4) - 2.0 * r(6) - r(7)
           + r(8) + 2.0 * r(9) + r(11) + r(13) - r(14))
    out_ref[...] = jnp.concatenate([w0, wa, wb, wab], axis=0)


def iv_and_hi(v):
    return v & jnp.int32(-65536)


_sc_mesh = plsc.VectorSubcoreMesh(core_axis_name="c", subcore_axis_name="s")


@functools.partial(
    pl.kernel,
    mesh=_sc_mesh,
    compiler_params=pltpu.CompilerParams(needs_layout_passes=False),
    out_type=jax.ShapeDtypeStruct((BATCH, OUT_DIM), jnp.float32),
    scratch_types=[
        pltpu.VMEM((NCHUNK,), jnp.int32),
        pltpu.VMEM((4 * CFS,), jnp.bfloat16),
        pltpu.VMEM((RGP * IN_DIM,), jnp.int32),
        pltpu.VMEM((2 * RG * OCW,), jnp.float32),
        pltpu.SemaphoreType.DMA,
        pltpu.SemaphoreType.DMA,
        pltpu.SemaphoreType.DMA,
    ],
)
def _sc_fn(x_hbm, idx_hbm, cf_hbm, out_hbm, idx_v, cf_v, x_v, o_v,
           sem_x, sem_o0, sem_o1):
    wid = lax.axis_index("s") * NC + lax.axis_index("c")
    nblk = wid % N_BLOCKS
    rblk = wid // N_BLOCKS
    n0 = nblk * NCHUNK
    row0 = rblk * ROWS_PER_TILE
    o_sems = (sem_o0, sem_o1)

    stage = [pltpu.make_async_copy(
        idx_hbm.at[pl.ds(n0, NCHUNK)], idx_v, sem_x)]
    for j in range(4):
        stage.append(pltpu.make_async_copy(
            cf_hbm.at[pl.ds(j * 2 * OUT_DIM + 2 * n0, CFS)],
            cf_v.at[pl.ds(j * CFS, CFS)], sem_x))
    for c in stage:
        c.start()
    for c in stage:
        c.wait()

    def group_body(g, carry):
        r0 = row0 + g * RG
        rp0 = row0 // 2 + g * RGP
        with jax.named_scope("xload"):
            xcps = [pltpu.make_async_copy(
                x_hbm.at[rp0 + q, :],
                x_v.at[pl.ds(q * IN_DIM, IN_DIM)], sem_x)
                for q in range(RGP)]
            for c in xcps:
                c.start()
            for c in xcps:
                c.wait()
        for oc in range(NOC):
            p = oc & 1
            col0 = n0 + oc * OCW

            def drain():
                for rr in range(RG):
                    pltpu.make_async_copy(
                        o_v.at[pl.ds((p * RG + rr) * OCW, OCW)],
                        out_hbm.at[r0 + rr, pl.ds(col0, OCW)],
                        o_sems[p]).wait()
            with jax.named_scope("drain"):
                if oc >= 2:
                    drain()
                else:
                    @pl.when(g > 0)
                    def _():
                        drain()

            sc_compute = jax.named_scope("compute")
            sc_compute.__enter__()

            @plsc.parallel_loop(0, NVEC, unroll=1)
            def vec_body(j):
                off = oc * OCW + j * 16
                off2 = 2 * off
                iv = idx_v[pl.ds(off, 16)]
                ia = iv & 0xFFFF
                ib = lax.shift_right_logical(iv, 16)
                w0 = cf_v[pl.ds(off2, 32)]
                wa = cf_v[pl.ds(CFS + off2, 32)]
                wb = cf_v[pl.ds(2 * CFS + off2, 32)]
                wab = cf_v[pl.ds(3 * CFS + off2, 32)]
                avs = [plsc.load_gather(x_v.at[pl.ds(q * IN_DIM, IN_DIM)],
                                        [ia]) for q in range(RGP)]
                bvs = [plsc.load_gather(x_v.at[pl.ds(q * IN_DIM, IN_DIM)],
                                        [ib]) for q in range(RGP)]
                for q in range(RGP):
                    a = plsc.bitcast(avs[q], jnp.bfloat16)
                    b = plsc.bitcast(bvs[q], jnp.bfloat16)
                    t = (wa + wab * b) * a + (wb * b + w0)
                    vi = plsc.bitcast(t, jnp.int32)
                    base = (p * RG + 2 * q) * OCW + j * 16
                    o_v[pl.ds(base, 16)] = plsc.bitcast(
                        lax.shift_left(vi, 16), jnp.float32)
                    o_v[pl.ds(base + OCW, 16)] = plsc.bitcast(
                        iv_and_hi(vi), jnp.float32)
            sc_compute.__exit__(None, None, None)
            with jax.named_scope("ostore"):
                for rr in range(RG):
                    pltpu.make_async_copy(
                        o_v.at[pl.ds((p * RG + rr) * OCW, OCW)],
                        out_hbm.at[r0 + rr, pl.ds(col0, OCW)],
                        o_sems[p]).start()
        return carry

    lax.fori_loop(0, NGROUPS, group_body, 0)

    r_last = row0 + (NGROUPS - 1) * RG
    for oc in (NOC - 2, NOC - 1):
        p = oc & 1
        for rr in range(RG):
            pltpu.make_async_copy(
                o_v.at[pl.ds((p * RG + rr) * OCW, OCW)],
                out_hbm.at[r_last + rr, pl.ds(n0 + oc * OCW, OCW)],
                o_sems[p]).wait()


def kernel(x, weight, indices):
    coeffs = pl.pallas_call(
        _coeff_body,
        out_shape=jax.ShapeDtypeStruct((4, OUT_DIM), jnp.float32),
    )(weight.T)
    cfd = jnp.repeat(coeffs.astype(jnp.bfloat16), 2, axis=1).reshape(-1)
    packed_idx = indices[0] | (indices[1] << 16)
    xb = jax.lax.bitcast_convert_type(
        x.astype(jnp.bfloat16), jnp.uint16).astype(jnp.uint32)
    xp = jax.lax.bitcast_convert_type(
        xb[0::2] | (xb[1::2] << 16), jnp.int32)
    return _sc_fn(xp, packed_idx, cfd)


# speedup vs baseline: 1.6590x; 1.6590x over previous
import functools

import jax
import jax.numpy as jnp
from jax import lax
from jax.experimental import pallas as pl
from jax.experimental.pallas import tpu as pltpu
from jax.experimental.pallas import tpu_sc as plsc

BATCH = 1024
IN_DIM = 4096
OUT_DIM = 16384

NC = 2
NS = 16
NW = NC * NS

N_BLOCKS = 2
R_BLOCKS = NW // N_BLOCKS
NCHUNK = OUT_DIM // N_BLOCKS
ROWS_PER_TILE = BATCH // R_BLOCKS
RG = 16
RGP = RG // 2
NGROUPS = ROWS_PER_TILE // RG
OCW = 1024
NOC = NCHUNK // OCW
NVEC = OCW // 16
CFS = 2 * NCHUNK


def _coeff_body(wt_ref, out_ref):
    w = wt_ref[...]
    m = jnp.max(w, axis=0, keepdims=True)
    e = jnp.exp(w - m)
    p = e / jnp.sum(e, axis=0, keepdims=True)

    def r(i):
        return p[i:i + 1, :]

    w0 = r(8) + r(9) + r(10) + r(11) + r(12) + r(13) + r(14) + r(15)
    wa = r(2) + r(3) + r(6) + r(7) - r(8) - r(9) - r(12) - r(13)
    wb = r(4) + r(5) + r(6) + r(7) - r(8) - r(9) - r(10) - r(11)
    wab = (r(1) - r(2) - r(4) - 2.0 * r(6) - r(7)
           + r(8) + 2.0 * r(9) + r(11) + r(13) - r(14))
    out_ref[...] = jnp.concatenate([w0, wa, wb, wab], axis=0)


def iv_and_hi(v):
    return v & jnp.int32(-65536)


_sc_mesh = plsc.VectorSubcoreMesh(core_axis_name="c", subcore_axis_name="s")


@functools.partial(
    pl.kernel,
    mesh=_sc_mesh,
    compiler_params=pltpu.CompilerParams(needs_layout_passes=False),
    out_type=jax.ShapeDtypeStruct((BATCH, OUT_DIM), jnp.float32),
    scratch_types=[
        pltpu.VMEM((NCHUNK,), jnp.int32),
        pltpu.VMEM((4 * CFS,), jnp.bfloat16),
        pltpu.VMEM((RGP * IN_DIM,), jnp.int32),
        pltpu.VMEM((2 * RG * OCW,), jnp.float32),
        pltpu.SemaphoreType.DMA,
        pltpu.SemaphoreType.DMA,
        pltpu.SemaphoreType.DMA,
    ],
)
def _sc_fn(x_hbm, idx_hbm, cf_hbm, out_hbm, idx_v, cf_v, x_v, o_v,
           sem_x, sem_o0, sem_o1):
    wid = lax.axis_index("s") * NC + lax.axis_index("c")
    nblk = wid % N_BLOCKS
    rblk = wid // N_BLOCKS
    n0 = nblk * NCHUNK
    row0p = rblk * (BATCH // 2 // R_BLOCKS)
    o_sems = (sem_o0, sem_o1)

    def orow(rp0, rr):
        return rp0 + rr // 2 + (rr % 2) * (BATCH // 2)

    stage = [pltpu.make_async_copy(
        idx_hbm.at[pl.ds(n0, NCHUNK)], idx_v, sem_x)]
    for j in range(4):
        stage.append(pltpu.make_async_copy(
            cf_hbm.at[pl.ds(j * 2 * OUT_DIM + 2 * n0, CFS)],
            cf_v.at[pl.ds(j * CFS, CFS)], sem_x))
    for c in stage:
        c.start()
    for c in stage:
        c.wait()

    def group_body(g, carry):
        rp0 = row0p + g * RGP
        with jax.named_scope("xload"):
            xcps = [pltpu.make_async_copy(
                x_hbm.at[rp0 + q, :],
                x_v.at[pl.ds(q * IN_DIM, IN_DIM)], sem_x)
                for q in range(RGP)]
            for c in xcps:
                c.start()
            for c in xcps:
                c.wait()
        for oc in range(NOC):
            p = oc & 1
            col0 = n0 + oc * OCW

            def drain():
                for rr in range(RG):
                    pltpu.make_async_copy(
                        o_v.at[pl.ds((p * RG + rr) * OCW, OCW)],
                        out_hbm.at[orow(rp0, rr), pl.ds(col0, OCW)],
                        o_sems[p]).wait()
            with jax.named_scope("drain"):
                if oc >= 2:
                    drain()
                else:
                    @pl.when(g > 0)
                    def _():
                        drain()

            sc_compute = jax.named_scope("compute")
            sc_compute.__enter__()

            @plsc.parallel_loop(0, NVEC, unroll=1)
            def vec_body(j):
                off = oc * OCW + j * 16
                off2 = 2 * off
                iv = idx_v[pl.ds(off, 16)]
                ia = iv & 0xFFFF
                ib = lax.shift_right_logical(iv, 16)
                w0 = cf_v[pl.ds(off2, 32)]
                wa = cf_v[pl.ds(CFS + off2, 32)]
                wb = cf_v[pl.ds(2 * CFS + off2, 32)]
                wab = cf_v[pl.ds(3 * CFS + off2, 32)]
                avs = [plsc.load_gather(x_v.at[pl.ds(q * IN_DIM, IN_DIM)],
                                        [ia]) for q in range(RGP)]
                bvs = [plsc.load_gather(x_v.at[pl.ds(q * IN_DIM, IN_DIM)],
                                        [ib]) for q in range(RGP)]
                for q in range(RGP):
                    a = plsc.bitcast(avs[q], jnp.bfloat16)
                    b = plsc.bitcast(bvs[q], jnp.bfloat16)
                    t = (wa + wab * b) * a + (wb * b + w0)
                    vi = plsc.bitcast(t, jnp.int32)
                    base = (p * RG + 2 * q) * OCW + j * 16
                    o_v[pl.ds(base, 16)] = plsc.bitcast(
                        lax.shift_left(vi, 16), jnp.float32)
                    o_v[pl.ds(base + OCW, 16)] = plsc.bitcast(
                        iv_and_hi(vi), jnp.float32)
            sc_compute.__exit__(None, None, None)
            with jax.named_scope("ostore"):
                for rr in range(RG):
                    pltpu.make_async_copy(
                        o_v.at[pl.ds((p * RG + rr) * OCW, OCW)],
                        out_hbm.at[orow(rp0, rr), pl.ds(col0, OCW)],
                        o_sems[p]).start()
        return carry

    lax.fori_loop(0, NGROUPS, group_body, 0)

    rp_last = row0p + (NGROUPS - 1) * RGP
    for oc in (NOC - 2, NOC - 1):
        p = oc & 1
        for rr in range(RG):
            pltpu.make_async_copy(
                o_v.at[pl.ds((p * RG + rr) * OCW, OCW)],
                out_hbm.at[orow(rp_last, rr), pl.ds(n0 + oc * OCW, OCW)],
                o_sems[p]).wait()


def kernel(x, weight, indices):
    coeffs = pl.pallas_call(
        _coeff_body,
        out_shape=jax.ShapeDtypeStruct((4, OUT_DIM), jnp.float32),
    )(weight.T)
    cfd = jnp.repeat(coeffs.astype(jnp.bfloat16), 2, axis=1).reshape(-1)
    packed_idx = indices[0] | (indices[1] << 16)
    xb = jax.lax.bitcast_convert_type(
        x.astype(jnp.bfloat16), jnp.uint16).astype(jnp.uint32)
    xp = jax.lax.bitcast_convert_type(
        xb[:BATCH // 2] | (xb[BATCH // 2:] << 16), jnp.int32)
    return _sc_fn(xp, packed_idx, cfd)


# speedup vs baseline: 2.2949x; 1.3833x over previous
import functools

import jax
import jax.numpy as jnp
from jax import lax
from jax.experimental import pallas as pl
from jax.experimental.pallas import tpu as pltpu
from jax.experimental.pallas import tpu_sc as plsc

BATCH = 1024
IN_DIM = 4096
OUT_DIM = 16384

NC = 2
NS = 16
NW = NC * NS

N_BLOCKS = 2
R_BLOCKS = NW // N_BLOCKS
NCHUNK = OUT_DIM // N_BLOCKS
ROWS_PER_TILE = BATCH // R_BLOCKS
RG = 16
RGP = RG // 2
NGROUPS = ROWS_PER_TILE // RG
OCW = 1024
NOC = NCHUNK // OCW
NVEC = OCW // 16
CFS = 2 * NCHUNK


def _coeff_body(wt_ref, out_ref):
    w = wt_ref[...]
    m = jnp.max(w, axis=0, keepdims=True)
    e = jnp.exp(w - m)
    p = e / jnp.sum(e, axis=0, keepdims=True)

    def r(i):
        return p[i:i + 1, :]

    w0 = r(8) + r(9) + r(10) + r(11) + r(12) + r(13) + r(14) + r(15)
    wa = r(2) + r(3) + r(6) + r(7) - r(8) - r(9) - r(12) - r(13)
    wb = r(4) + r(5) + r(6) + r(7) - r(8) - r(9) - r(10) - r(11)
    wab = (r(1) - r(2) - r(4) - 2.0 * r(6) - r(7)
           + r(8) + 2.0 * r(9) + r(11) + r(13) - r(14))
    out_ref[...] = jnp.concatenate([w0, wa, wb, wab], axis=0)


def iv_and_hi(v):
    return v & jnp.int32(-65536)


_sc_mesh = plsc.VectorSubcoreMesh(core_axis_name="c", subcore_axis_name="s")


@functools.partial(
    pl.kernel,
    mesh=_sc_mesh,
    compiler_params=pltpu.CompilerParams(needs_layout_passes=False),
    out_type=jax.ShapeDtypeStruct((BATCH, OUT_DIM), jnp.float32),
    scratch_types=[
        pltpu.VMEM((NCHUNK,), jnp.int32),
        pltpu.VMEM((4 * NCHUNK,), jnp.float32),
        pltpu.VMEM((RGP * IN_DIM,), jnp.int32),
        pltpu.VMEM((2 * RG * OCW,), jnp.float32),
        pltpu.SemaphoreType.DMA,
        pltpu.SemaphoreType.DMA,
        pltpu.SemaphoreType.DMA,
    ],
)
def _sc_fn(x_hbm, idx_hbm, cf_hbm, out_hbm, idx_v, cf_v, x_v, o_v,
           sem_x, sem_o0, sem_o1):
    wid = lax.axis_index("s") * NC + lax.axis_index("c")
    nblk = wid % N_BLOCKS
    rblk = wid // N_BLOCKS
    n0 = nblk * NCHUNK
    row0p = rblk * (BATCH // 2 // R_BLOCKS)
    o_sems = (sem_o0, sem_o1)

    def orow(rp0, rr):
        return rp0 + rr // 2 + (rr % 2) * (BATCH // 2)

    stage = [pltpu.make_async_copy(
        idx_hbm.at[pl.ds(n0, NCHUNK)], idx_v, sem_x)]
    for j in range(4):
        stage.append(pltpu.make_async_copy(
            cf_hbm.at[j, pl.ds(n0, NCHUNK)],
            cf_v.at[pl.ds(j * NCHUNK, NCHUNK)], sem_x))
    for c in stage:
        c.start()
    for c in stage:
        c.wait()

    def group_body(g, carry):
        rp0 = row0p + g * RGP
        with jax.named_scope("xload"):
            xcps = [pltpu.make_async_copy(
                x_hbm.at[rp0 + q, :],
                x_v.at[pl.ds(q * IN_DIM, IN_DIM)], sem_x)
                for q in range(RGP)]
            for c in xcps:
                c.start()
            for c in xcps:
                c.wait()
        for oc in range(NOC):
            p = oc & 1
            col0 = n0 + oc * OCW

            def drain():
                for rr in range(RG):
                    pltpu.make_async_copy(
                        o_v.at[pl.ds((p * RG + rr) * OCW, OCW)],
                        out_hbm.at[orow(rp0, rr), pl.ds(col0, OCW)],
                        o_sems[p]).wait()
            with jax.named_scope("drain"):
                if oc >= 2:
                    drain()
                else:
                    @pl.when(g > 0)
                    def _():
                        drain()

            sc_compute = jax.named_scope("compute")
            sc_compute.__enter__()

            @plsc.parallel_loop(0, NVEC, unroll=1)
            def vec_body(j):
                off = oc * OCW + j * 16
                iv = idx_v[pl.ds(off, 16)]
                ia = iv & 0xFFFF
                ib = lax.shift_right_logical(iv, 16)

                def dup(plane):
                    w = cf_v[pl.ds(plane * NCHUNK + off, 16)]
                    return plsc.pack(w, w, format=plsc.PackFormat.INTERLEAVED)

                w0, wa, wb, wab = dup(0), dup(1), dup(2), dup(3)
                avs = [plsc.load_gather(x_v.at[pl.ds(q * IN_DIM, IN_DIM)],
                                        [ia]) for q in range(RGP)]
                bvs = [plsc.load_gather(x_v.at[pl.ds(q * IN_DIM, IN_DIM)],
                                        [ib]) for q in range(RGP)]
                for q in range(RGP):
                    a = plsc.bitcast(avs[q], jnp.bfloat16)
                    b = plsc.bitcast(bvs[q], jnp.bfloat16)
                    t = (wa + wab * b) * a + (wb * b + w0)
                    vi = plsc.bitcast(t, jnp.int32)
                    base = (p * RG + 2 * q) * OCW + j * 16
                    o_v[pl.ds(base, 16)] = plsc.bitcast(
                        lax.shift_left(vi, 16), jnp.float32)
                    o_v[pl.ds(base + OCW, 16)] = plsc.bitcast(
                        iv_and_hi(vi), jnp.float32)
            sc_compute.__exit__(None, None, None)
            with jax.named_scope("ostore"):
                for rr in range(RG):
                    pltpu.make_async_copy(
                        o_v.at[pl.ds((p * RG + rr) * OCW, OCW)],
                        out_hbm.at[orow(rp0, rr), pl.ds(col0, OCW)],
                        o_sems[p]).start()
        return carry

    lax.fori_loop(0, NGROUPS, group_body, 0)

    rp_last = row0p + (NGROUPS - 1) * RGP
    for oc in (NOC - 2, NOC - 1):
        p = oc & 1
        for rr in range(RG):
            pltpu.make_async_copy(
                o_v.at[pl.ds((p * RG + rr) * OCW, OCW)],
                out_hbm.at[orow(rp_last, rr), pl.ds(n0 + oc * OCW, OCW)],
                o_sems[p]).wait()


def kernel(x, weight, indices):
    coeffs = pl.pallas_call(
        _coeff_body,
        out_shape=jax.ShapeDtypeStruct((4, OUT_DIM), jnp.float32),
    )(weight.T)
    packed_idx = indices[0] | (indices[1] << 16)
    xb = jax.lax.bitcast_convert_type(
        x.astype(jnp.bfloat16), jnp.uint16).astype(jnp.uint32)
    xp = jax.lax.bitcast_convert_type(
        xb[:BATCH // 2] | (xb[BATCH // 2:] << 16), jnp.int32)
    return _sc_fn(xp, packed_idx, coeffs)


# speedup vs baseline: 2.3482x; 1.0233x over previous
import functools

import jax
import jax.numpy as jnp
from jax import lax
from jax.experimental import pallas as pl
from jax.experimental.pallas import tpu as pltpu
from jax.experimental.pallas import tpu_sc as plsc

BATCH = 1024
IN_DIM = 4096
OUT_DIM = 16384

NC = 2
NS = 16
NW = NC * NS

N_BLOCKS = 2
R_BLOCKS = NW // N_BLOCKS
NCHUNK = OUT_DIM // N_BLOCKS
ROWS_PER_TILE = BATCH // R_BLOCKS
RG = 16
RGP = RG // 2
NGROUPS = ROWS_PER_TILE // RG
OCW = 512
NOC = NCHUNK // OCW
NVEC = OCW // 16
XBUF = RGP * IN_DIM


def _coeff_body(wt_ref, out_ref):
    w = wt_ref[...]
    m = jnp.max(w, axis=0, keepdims=True)
    e = jnp.exp(w - m)
    p = e / jnp.sum(e, axis=0, keepdims=True)

    def r(i):
        return p[i:i + 1, :]

    w0 = r(8) + r(9) + r(10) + r(11) + r(12) + r(13) + r(14) + r(15)
    wa = r(2) + r(3) + r(6) + r(7) - r(8) - r(9) - r(12) - r(13)
    wb = r(4) + r(5) + r(6) + r(7) - r(8) - r(9) - r(10) - r(11)
    wab = (r(1) - r(2) - r(4) - 2.0 * r(6) - r(7)
           + r(8) + 2.0 * r(9) + r(11) + r(13) - r(14))
    out_ref[...] = jnp.concatenate([w0, wa, wb, wab], axis=0)


def iv_and_hi(v):
    return v & jnp.int32(-65536)


_sc_mesh = plsc.VectorSubcoreMesh(core_axis_name="c", subcore_axis_name="s")


@functools.partial(
    pl.kernel,
    mesh=_sc_mesh,
    compiler_params=pltpu.CompilerParams(needs_layout_passes=False),
    out_type=jax.ShapeDtypeStruct((BATCH, OUT_DIM), jnp.float32),
    scratch_types=[
        pltpu.VMEM((NCHUNK,), jnp.int32),
        pltpu.VMEM((4 * NCHUNK,), jnp.float32),
        pltpu.VMEM((2 * RGP * IN_DIM,), jnp.int32),
        pltpu.VMEM((2 * RG * OCW,), jnp.float32),
        pltpu.SemaphoreType.DMA,
        pltpu.SemaphoreType.DMA,
        pltpu.SemaphoreType.DMA,
    ],
)
def _sc_fn(x_hbm, idx_hbm, cf_hbm, out_hbm, idx_v, cf_v, x_v, o_v,
           sem_x, sem_o0, sem_o1):
    wid = lax.axis_index("s") * NC + lax.axis_index("c")
    nblk = wid % N_BLOCKS
    rblk = wid // N_BLOCKS
    n0 = nblk * NCHUNK
    row0p = rblk * (BATCH // 2 // R_BLOCKS)
    o_sems = (sem_o0, sem_o1)

    def orow(rp0, rr):
        return rp0 + rr // 2 + (rr % 2) * (BATCH // 2)

    stage = [pltpu.make_async_copy(
        idx_hbm.at[pl.ds(n0, NCHUNK)], idx_v, sem_x)]
    for j in range(4):
        stage.append(pltpu.make_async_copy(
            cf_hbm.at[j, pl.ds(n0, NCHUNK)],
            cf_v.at[pl.ds(j * NCHUNK, NCHUNK)], sem_x))
    for c in stage:
        c.start()
    for c in stage:
        c.wait()

    def fire_xload(g):
        rp0 = row0p + g * RGP
        xoff = (g % 2) * XBUF
        for q in range(RGP):
            pltpu.make_async_copy(
                x_hbm.at[rp0 + q, :],
                x_v.at[pl.ds(xoff + q * IN_DIM, IN_DIM)], sem_x).start()

    fire_xload(0)

    def group_body(g, carry):
        rp0 = row0p + g * RGP
        xoff = (g % 2) * XBUF
        for q in range(RGP):
            pltpu.make_async_copy(
                x_hbm.at[rp0 + q, :],
                x_v.at[pl.ds(xoff + q * IN_DIM, IN_DIM)], sem_x).wait()

        @pl.when(g < NGROUPS - 1)
        def _():
            fire_xload(g + 1)
        for oc in range(NOC):
            p = oc & 1
            col0 = n0 + oc * OCW

            def drain():
                for rr in range(RG):
                    pltpu.make_async_copy(
                        o_v.at[pl.ds((p * RG + rr) * OCW, OCW)],
                        out_hbm.at[orow(rp0, rr), pl.ds(col0, OCW)],
                        o_sems[p]).wait()
            if oc >= 2:
                drain()
            else:
                @pl.when(g > 0)
                def _():
                    drain()

            @plsc.parallel_loop(0, NVEC, unroll=1)
            def vec_body(j):
                off = oc * OCW + j * 16
                iv = idx_v[pl.ds(off, 16)]
                ia = iv & 0xFFFF
                ib = lax.shift_right_logical(iv, 16)

                def dup(plane):
                    w = cf_v[pl.ds(plane * NCHUNK + off, 16)]
                    return plsc.pack(w, w, format=plsc.PackFormat.INTERLEAVED)

                w0, wa, wb, wab = dup(0), dup(1), dup(2), dup(3)
                avs = [plsc.load_gather(
                    x_v.at[pl.ds(xoff + q * IN_DIM, IN_DIM)],
                    [ia]) for q in range(RGP)]
                bvs = [plsc.load_gather(
                    x_v.at[pl.ds(xoff + q * IN_DIM, IN_DIM)],
                    [ib]) for q in range(RGP)]
                for q in range(RGP):
                    a = plsc.bitcast(avs[q], jnp.bfloat16)
                    b = plsc.bitcast(bvs[q], jnp.bfloat16)
                    t = (wa + wab * b) * a + (wb * b + w0)
                    vi = plsc.bitcast(t, jnp.int32)
                    base = (p * RG + 2 * q) * OCW + j * 16
                    o_v[pl.ds(base, 16)] = plsc.bitcast(
                        lax.shift_left(vi, 16), jnp.float32)
                    o_v[pl.ds(base + OCW, 16)] = plsc.bitcast(
                        iv_and_hi(vi), jnp.float32)
            for rr in range(RG):
                pltpu.make_async_copy(
                    o_v.at[pl.ds((p * RG + rr) * OCW, OCW)],
                    out_hbm.at[orow(rp0, rr), pl.ds(col0, OCW)],
                    o_sems[p]).start()
        return carry

    lax.fori_loop(0, NGROUPS, group_body, 0)

    rp_last = row0p + (NGROUPS - 1) * RGP
    for oc in (NOC - 2, NOC - 1):
        p = oc & 1
        for rr in range(RG):
            pltpu.make_async_copy(
                o_v.at[pl.ds((p * RG + rr) * OCW, OCW)],
                out_hbm.at[orow(rp_last, rr), pl.ds(n0 + oc * OCW, OCW)],
                o_sems[p]).wait()


def kernel(x, weight, indices):
    coeffs = pl.pallas_call(
        _coeff_body,
        out_shape=jax.ShapeDtypeStruct((4, OUT_DIM), jnp.float32),
    )(weight.T)
    packed_idx = indices[0] | (indices[1] << 16)
    xb = jax.lax.bitcast_convert_type(
        x.astype(jnp.bfloat16), jnp.uint16).astype(jnp.uint32)
    xp = jax.lax.bitcast_convert_type(
        xb[:BATCH // 2] | (xb[BATCH // 2:] << 16), jnp.int32)
    return _sc_fn(xp, packed_idx, coeffs)


# speedup vs baseline: 2.4117x; 1.0270x over previous
import functools

import jax
import jax.numpy as jnp
from jax import lax
from jax.experimental import pallas as pl
from jax.experimental.pallas import tpu as pltpu
from jax.experimental.pallas import tpu_sc as plsc

BATCH = 1024
IN_DIM = 4096
OUT_DIM = 16384

NC = 2
NS = 16
NW = NC * NS

N_BLOCKS = 2
R_BLOCKS = NW // N_BLOCKS
NCHUNK = OUT_DIM // N_BLOCKS
ROWS_PER_TILE = BATCH // R_BLOCKS
RG = 16
RGP = RG // 2
NGROUPS = ROWS_PER_TILE // RG
OCW = 512
NOC = NCHUNK // OCW
NVEC = OCW // 16
XBUF = RGP * IN_DIM


def _prep_body(xt_ref, xb_ref, wt_ref, idx_ref, xp_ref, cf_ref, pidx_ref):
    lo = jax.lax.bitcast_convert_type(
        xt_ref[...].astype(jnp.bfloat16), jnp.uint16).astype(jnp.uint32)
    hi = jax.lax.bitcast_convert_type(
        xb_ref[...].astype(jnp.bfloat16), jnp.uint16).astype(jnp.uint32)
    xp_ref[...] = jax.lax.bitcast_convert_type(lo | (hi << 16), jnp.int32)

    @pl.when(pl.program_id(0) == 0)
    def _():
        w = wt_ref[...]
        m = jnp.max(w, axis=0, keepdims=True)
        e = jnp.exp(w - m)
        p = e / jnp.sum(e, axis=0, keepdims=True)

        def r(i):
            return p[i:i + 1, :]

        w0 = r(8) + r(9) + r(10) + r(11) + r(12) + r(13) + r(14) + r(15)
        wa = r(2) + r(3) + r(6) + r(7) - r(8) - r(9) - r(12) - r(13)
        wb = r(4) + r(5) + r(6) + r(7) - r(8) - r(9) - r(10) - r(11)
        wab = (r(1) - r(2) - r(4) - 2.0 * r(6) - r(7)
               + r(8) + 2.0 * r(9) + r(11) + r(13) - r(14))
        cf_ref[...] = jnp.concatenate([w0, wa, wb, wab], axis=0)
        pidx_ref[...] = idx_ref[0:1, :] | (idx_ref[1:2, :] << 16)


def iv_and_hi(v):
    return v & jnp.int32(-65536)


_sc_mesh = plsc.VectorSubcoreMesh(core_axis_name="c", subcore_axis_name="s")


@functools.partial(
    pl.kernel,
    mesh=_sc_mesh,
    compiler_params=pltpu.CompilerParams(needs_layout_passes=False),
    out_type=jax.ShapeDtypeStruct((BATCH, OUT_DIM), jnp.float32),
    scratch_types=[
        pltpu.VMEM((NCHUNK,), jnp.int32),
        pltpu.VMEM((4 * NCHUNK,), jnp.float32),
        pltpu.VMEM((2 * RGP * IN_DIM,), jnp.int32),
        pltpu.VMEM((2 * RG * OCW,), jnp.float32),
        pltpu.SemaphoreType.DMA,
        pltpu.SemaphoreType.DMA,
        pltpu.SemaphoreType.DMA,
    ],
)
def _sc_fn(x_hbm, idx_hbm, cf_hbm, out_hbm, idx_v, cf_v, x_v, o_v,
           sem_x, sem_o0, sem_o1):
    wid = lax.axis_index("s") * NC + lax.axis_index("c")
    nblk = wid % N_BLOCKS
    rblk = wid // N_BLOCKS
    n0 = nblk * NCHUNK
    row0p = rblk * (BATCH // 2 // R_BLOCKS)
    o_sems = (sem_o0, sem_o1)

    def orow(rp0, rr):
        return rp0 + rr // 2 + (rr % 2) * (BATCH // 2)

    stage = [pltpu.make_async_copy(
        idx_hbm.at[0, pl.ds(n0, NCHUNK)], idx_v, sem_x)]
    for j in range(4):
        stage.append(pltpu.make_async_copy(
            cf_hbm.at[j, pl.ds(n0, NCHUNK)],
            cf_v.at[pl.ds(j * NCHUNK, NCHUNK)], sem_x))
    for c in stage:
        c.start()
    for c in stage:
        c.wait()

    def fire_xload(g):
        rp0 = row0p + g * RGP
        xoff = (g % 2) * XBUF
        for q in range(RGP):
            pltpu.make_async_copy(
                x_hbm.at[rp0 + q, :],
                x_v.at[pl.ds(xoff + q * IN_DIM, IN_DIM)], sem_x).start()

    fire_xload(0)

    def group_body(g, carry):
        rp0 = row0p + g * RGP
        xoff = (g % 2) * XBUF
        for q in range(RGP):
            pltpu.make_async_copy(
                x_hbm.at[rp0 + q, :],
                x_v.at[pl.ds(xoff + q * IN_DIM, IN_DIM)], sem_x).wait()

        @pl.when(g < NGROUPS - 1)
        def _():
            fire_xload(g + 1)
        for oc in range(NOC):
            p = oc & 1
            col0 = n0 + oc * OCW

            def drain():
                for rr in range(RG):
                    pltpu.make_async_copy(
                        o_v.at[pl.ds((p * RG + rr) * OCW, OCW)],
                        out_hbm.at[orow(rp0, rr), pl.ds(col0, OCW)],
                        o_sems[p]).wait()
            if oc >= 2:
                drain()
            else:
                @pl.when(g > 0)
                def _():
                    drain()

            @plsc.parallel_loop(0, NVEC, unroll=1)
            def vec_body(j):
                off = oc * OCW + j * 16
                iv = idx_v[pl.ds(off, 16)]
                ia = iv & 0xFFFF
                ib = lax.shift_right_logical(iv, 16)

                def dup(plane):
                    w = cf_v[pl.ds(plane * NCHUNK + off, 16)]
                    return plsc.pack(w, w, format=plsc.PackFormat.INTERLEAVED)

                w0, wa, wb, wab = dup(0), dup(1), dup(2), dup(3)
                avs = [plsc.load_gather(
                    x_v.at[pl.ds(xoff + q * IN_DIM, IN_DIM)],
                    [ia]) for q in range(RGP)]
                bvs = [plsc.load_gather(
                    x_v.at[pl.ds(xoff + q * IN_DIM, IN_DIM)],
                    [ib]) for q in range(RGP)]
                for q in range(RGP):
                    a = plsc.bitcast(avs[q], jnp.bfloat16)
                    b = plsc.bitcast(bvs[q], jnp.bfloat16)
                    t = (wa + wab * b) * a + (wb * b + w0)
                    vi = plsc.bitcast(t, jnp.int32)
                    base = (p * RG + 2 * q) * OCW + j * 16
                    o_v[pl.ds(base, 16)] = plsc.bitcast(
                        lax.shift_left(vi, 16), jnp.float32)
                    o_v[pl.ds(base + OCW, 16)] = plsc.bitcast(
                        iv_and_hi(vi), jnp.float32)
            for rr in range(RG):
                pltpu.make_async_copy(
                    o_v.at[pl.ds((p * RG + rr) * OCW, OCW)],
                    out_hbm.at[orow(rp0, rr), pl.ds(col0, OCW)],
                    o_sems[p]).start()
        return carry

    lax.fori_loop(0, NGROUPS, group_body, 0)

    rp_last = row0p + (NGROUPS - 1) * RGP
    for oc in (NOC - 2, NOC - 1):
        p = oc & 1
        for rr in range(RG):
            pltpu.make_async_copy(
                o_v.at[pl.ds((p * RG + rr) * OCW, OCW)],
                out_hbm.at[orow(rp_last, rr), pl.ds(n0 + oc * OCW, OCW)],
                o_sems[p]).wait()


def kernel(x, weight, indices):
    xp, coeffs, pidx = pl.pallas_call(
        _prep_body,
        grid=(8,),
        in_specs=[
            pl.BlockSpec((64, IN_DIM), lambda i: (i, 0)),
            pl.BlockSpec((64, IN_DIM), lambda i: (i + 8, 0)),
            pl.BlockSpec((16, OUT_DIM), lambda i: (0, 0)),
            pl.BlockSpec((2, OUT_DIM), lambda i: (0, 0)),
        ],
        out_specs=[
            pl.BlockSpec((64, IN_DIM), lambda i: (i, 0)),
            pl.BlockSpec((4, OUT_DIM), lambda i: (0, 0)),
            pl.BlockSpec((1, OUT_DIM), lambda i: (0, 0)),
        ],
        out_shape=[
            jax.ShapeDtypeStruct((BATCH // 2, IN_DIM), jnp.int32),
            jax.ShapeDtypeStruct((4, OUT_DIM), jnp.float32),
            jax.ShapeDtypeStruct((1, OUT_DIM), jnp.int32),
        ],
    )(x, x, weight.T, indices)
    return _sc_fn(xp, pidx, coeffs)


# speedup vs baseline: 2.5518x; 1.0581x over previous
import functools

import jax
import jax.numpy as jnp
from jax import lax
from jax.experimental import pallas as pl
from jax.experimental.pallas import tpu as pltpu
from jax.experimental.pallas import tpu_sc as plsc

BATCH = 1024
IN_DIM = 4096
OUT_DIM = 16384

NC = 2
NS = 16
NW = NC * NS

N_BLOCKS = 2
R_BLOCKS = NW // N_BLOCKS
NCHUNK = OUT_DIM // N_BLOCKS
ROWS_PER_TILE = BATCH // R_BLOCKS
RG = 16
RGP = RG // 2
NGROUPS = ROWS_PER_TILE // RG
OCW = 512
NOC = NCHUNK // OCW
NVEC = OCW // 16
XBUF = RGP * IN_DIM


def _prep_body(xt_ref, xb_ref, wt_ref, idx_ref, xp_ref, cf_ref, pidx_ref):
    lo = jax.lax.bitcast_convert_type(
        xt_ref[...].astype(jnp.bfloat16), jnp.uint16).astype(jnp.uint32)
    hi = jax.lax.bitcast_convert_type(
        xb_ref[...].astype(jnp.bfloat16), jnp.uint16).astype(jnp.uint32)
    xp_ref[...] = jax.lax.bitcast_convert_type(lo | (hi << 16), jnp.int32)

    @pl.when(pl.program_id(0) == 0)
    def _():
        w = wt_ref[...]
        m = jnp.max(w, axis=0, keepdims=True)
        e = jnp.exp(w - m)
        p = e / jnp.sum(e, axis=0, keepdims=True)

        def r(i):
            return p[i:i + 1, :]

        w0 = r(8) + r(9) + r(10) + r(11) + r(12) + r(13) + r(14) + r(15)
        wa = r(2) + r(3) + r(6) + r(7) - r(8) - r(9) - r(12) - r(13)
        wb = r(4) + r(5) + r(6) + r(7) - r(8) - r(9) - r(10) - r(11)
        wab = (r(1) - r(2) - r(4) - 2.0 * r(6) - r(7)
               + r(8) + 2.0 * r(9) + r(11) + r(13) - r(14))
        cf_ref[...] = jnp.concatenate([w0, wa, wb, wab], axis=0)
        pidx_ref[...] = idx_ref[0:1, :] | (idx_ref[1:2, :] << 16)


def iv_and_hi(v):
    return v & jnp.int32(-65536)


_sc_mesh = plsc.VectorSubcoreMesh(core_axis_name="c", subcore_axis_name="s")


@functools.partial(
    pl.kernel,
    mesh=_sc_mesh,
    compiler_params=pltpu.CompilerParams(needs_layout_passes=False),
    out_type=jax.ShapeDtypeStruct((BATCH, OUT_DIM), jnp.float32),
    scratch_types=[
        pltpu.VMEM((NCHUNK,), jnp.int32),
        pltpu.VMEM((4 * NCHUNK,), jnp.float32),
        pltpu.VMEM((2 * RGP * IN_DIM,), jnp.int32),
        pltpu.VMEM((2 * RG * OCW,), jnp.float32),
        pltpu.SemaphoreType.DMA,
        pltpu.SemaphoreType.DMA,
        pltpu.SemaphoreType.DMA,
    ],
)
def _sc_fn(x_hbm, idx_hbm, cf_hbm, out_hbm, idx_v, cf_v, x_v, o_v,
           sem_x, sem_o0, sem_o1):
    wid = lax.axis_index("s") * NC + lax.axis_index("c")
    nblk = wid % N_BLOCKS
    rblk = wid // N_BLOCKS
    n0 = nblk * NCHUNK
    row0p = rblk * (BATCH // 2 // R_BLOCKS)
    o_sems = (sem_o0, sem_o1)

    def fire_xload(g):
        rp0 = row0p + g * RGP
        xoff = (g % 2) * XBUF
        for q in range(RGP):
            pltpu.make_async_copy(
                x_hbm.at[rp0 + q, :],
                x_v.at[pl.ds(xoff + q * IN_DIM, IN_DIM)], sem_x).start()

    def drain_chunk(p, rp0):
        pltpu.make_async_copy(
            o_v.at[pl.ds(p * RG * OCW, RG * OCW)],
            out_hbm.at[rp0, pl.ds(0, RG * OCW)], o_sems[p]).wait()

    def orow(rp0, rr):
        return rp0 + rr // 2 + (rr % 2) * (BATCH // 2)

    stage = [pltpu.make_async_copy(
        idx_hbm.at[0, pl.ds(n0, NCHUNK)], idx_v, sem_x)]
    for j in range(4):
        stage.append(pltpu.make_async_copy(
            cf_hbm.at[j, pl.ds(n0, NCHUNK)],
            cf_v.at[pl.ds(j * NCHUNK, NCHUNK)], sem_x))
    for c in stage:
        c.start()
    fire_xload(0)
    for c in stage:
        c.wait()

    def group_body(g, carry):
        rp0 = row0p + g * RGP
        xoff = (g % 2) * XBUF
        for q in range(RGP):
            pltpu.make_async_copy(
                x_hbm.at[rp0 + q, :],
                x_v.at[pl.ds(xoff + q * IN_DIM, IN_DIM)], sem_x).wait()

        @pl.when(g < NGROUPS - 1)
        def _():
            fire_xload(g + 1)
        for oc in range(NOC):
            p = oc & 1
            col0 = n0 + oc * OCW

            if oc >= 2:
                drain_chunk(p, rp0)
            else:
                @pl.when(g > 0)
                def _():
                    drain_chunk(p, rp0)

            @plsc.parallel_loop(0, NVEC, unroll=1)
            def vec_body(j):
                off = oc * OCW + j * 16
                iv = idx_v[pl.ds(off, 16)]
                ia = iv & 0xFFFF
                ib = lax.shift_right_logical(iv, 16)

                def dup(plane):
                    w = cf_v[pl.ds(plane * NCHUNK + off, 16)]
                    return plsc.pack(w, w, format=plsc.PackFormat.INTERLEAVED)

                w0, wa, wb, wab = dup(0), dup(1), dup(2), dup(3)
                avs = [plsc.load_gather(
                    x_v.at[pl.ds(xoff + q * IN_DIM, IN_DIM)],
                    [ia]) for q in range(RGP)]
                bvs = [plsc.load_gather(
                    x_v.at[pl.ds(xoff + q * IN_DIM, IN_DIM)],
                    [ib]) for q in range(RGP)]
                for q in range(RGP):
                    a = plsc.bitcast(avs[q], jnp.bfloat16)
                    b = plsc.bitcast(bvs[q], jnp.bfloat16)
                    t = (wa + wab * b) * a + (wb * b + w0)
                    vi = plsc.bitcast(t, jnp.int32)
                    base = (p * RG + 2 * q) * OCW + j * 16
                    o_v[pl.ds(base, 16)] = plsc.bitcast(
                        lax.shift_left(vi, 16), jnp.float32)
                    o_v[pl.ds(base + OCW, 16)] = plsc.bitcast(
                        iv_and_hi(vi), jnp.float32)
            for rr in range(RG):
                pltpu.make_async_copy(
                    o_v.at[pl.ds((p * RG + rr) * OCW, OCW)],
                    out_hbm.at[orow(rp0, rr), pl.ds(col0, OCW)],
                    o_sems[p]).start()
        return carry

    lax.fori_loop(0, NGROUPS, group_body, 0)

    rp_last = row0p + (NGROUPS - 1) * RGP
    drain_chunk(0, rp_last)
    drain_chunk(1, rp_last)


def kernel(x, weight, indices):
    xp, coeffs, pidx = pl.pallas_call(
        _prep_body,
        grid=(8,),
        in_specs=[
            pl.BlockSpec((64, IN_DIM), lambda i: (i, 0)),
            pl.BlockSpec((64, IN_DIM), lambda i: (i + 8, 0)),
            pl.BlockSpec((16, OUT_DIM), lambda i: (0, 0)),
            pl.BlockSpec((2, OUT_DIM), lambda i: (0, 0)),
        ],
        out_specs=[
            pl.BlockSpec((64, IN_DIM), lambda i: (i, 0)),
            pl.BlockSpec((4, OUT_DIM), lambda i: (0, 0)),
            pl.BlockSpec((1, OUT_DIM), lambda i: (0, 0)),
        ],
        out_shape=[
            jax.ShapeDtypeStruct((BATCH // 2, IN_DIM), jnp.int32),
            jax.ShapeDtypeStruct((4, OUT_DIM), jnp.float32),
            jax.ShapeDtypeStruct((1, OUT_DIM), jnp.int32),
        ],
    )(x, x, weight.T, indices)
    return _sc_fn(xp, pidx, coeffs)
